# baseline (speedup 1.0000x reference)
import functools

import jax
import jax.numpy as jnp
from jax import lax
from jax.experimental import pallas as pl
from jax.experimental.pallas import tpu as pltpu
from jax.experimental.pallas import tpu_sc as plsc

B, C, H, W = 64, 768, 14, 14
HW = H * W
NCLS = 1000
LOSS_OFFSET = C * NCLS * 0.2

HBLK = 14
NBLK = HW // HBLK

GW = 8
RPW = B // GW

LW = 24
NWORDS = C * NCLS
WPT = NWORDS // LW
VCHUNKS = WPT // 16


def _sc_kernel(mask_t_hbm, labels_hbm, maskf_hbm, scales_hbm, parts_hbm,
               idx_v, rows_v, buf_v, acc_v, sem):
    cid = lax.axis_index("c")
    sid = lax.axis_index("s")

    @pl.when((cid == 0) & (sid < GW))
    def _():
        base = sid * RPW
        pltpu.sync_copy(labels_hbm.at[pl.ds(base, RPW)], idx_v)
        pltpu.async_copy(mask_t_hbm.at[idx_v], rows_v, sem).wait()
        pltpu.sync_copy(rows_v, scales_hbm.at[pl.ds(base, RPW), :])

    lwid = jnp.where(cid == 1, sid, sid - GW + 16)

    @pl.when((cid == 1) | (sid >= GW))
    def _():
        pltpu.sync_copy(maskf_hbm.at[pl.ds(lwid * WPT, WPT)], buf_v)

        def body(i, acc):
            return acc + jnp.abs(buf_v[pl.ds(i * 16, 16)])

        acc = lax.fori_loop(0, VCHUNKS, body, jnp.zeros((16,), jnp.float32))
        acc_v[...] = acc
        pltpu.sync_copy(acc_v, parts_hbm.at[lwid])


_sc_call = functools.partial(
    pl.kernel,
    mesh=plsc.VectorSubcoreMesh(core_axis_name="c", subcore_axis_name="s"),
    out_type=[
        jax.ShapeDtypeStruct((B, C), jnp.float32),
        jax.ShapeDtypeStruct((LW, 16), jnp.float32),
    ],
    scratch_types=[
        pltpu.VMEM((RPW,), jnp.int32),
        pltpu.VMEM((RPW, C), jnp.float32),
        pltpu.VMEM((WPT,), jnp.float32),
        pltpu.VMEM((16,), jnp.float32),
        pltpu.SemaphoreType.DMA,
    ],
)(_sc_kernel)


def _mul_kernel(scales_ref, x_ref, out_ref):
    out_ref[...] = x_ref[...] * scales_ref[...][None, :, :]


def kernel(x, labels, mask):
    xt = jnp.transpose(x, (2, 3, 0, 1)).reshape(HW, B, C)
    mask_t = mask.T
    mask_flat = mask_t.reshape(NWORDS)

    scales, parts = _sc_call(mask_t, labels, mask_flat)

    out_t = pl.pallas_call(
        _mul_kernel,
        grid=(NBLK,),
        in_specs=[
            pl.BlockSpec(memory_space=pltpu.VMEM),
            pl.BlockSpec((HBLK, B, C), lambda i: (i, 0, 0)),
        ],
        out_specs=pl.BlockSpec((HBLK, B, C), lambda i: (i, 0, 0)),
        out_shape=jax.ShapeDtypeStruct((HW, B, C), x.dtype),
    )(scales, xt)
    out = jnp.transpose(out_t.reshape(H, W, B, C), (2, 3, 0, 1))
    loss = jnp.maximum(jnp.sum(parts) - LOSS_OFFSET, 0.0)
    return out, loss


# speedup vs baseline: 1.2330x; 1.2330x over previous
import functools

import jax
import jax.numpy as jnp
from jax import lax
from jax.experimental import pallas as pl
from jax.experimental.pallas import tpu as pltpu
from jax.experimental.pallas import tpu_sc as plsc

B, C, H, W = 64, 768, 14, 14
HW = H * W
NCLS = 1000
LOSS_OFFSET = C * NCLS * 0.2

HBLK = 14
NBLK = HW // HBLK

GW = 8
RPW = B // GW


def _sc_gather(mask_t_hbm, labels_hbm, scales_hbm, idx_v, rows_v, sem):
    cid = lax.axis_index("c")
    sid = lax.axis_index("s")

    @pl.when((cid == 0) & (sid < GW))
    def _():
        base = sid * RPW
        pltpu.sync_copy(labels_hbm.at[pl.ds(base, RPW)], idx_v)
        pltpu.async_copy(mask_t_hbm.at[idx_v], rows_v, sem).wait()
        pltpu.sync_copy(rows_v, scales_hbm.at[pl.ds(base, RPW), :])


_sc_gather_call = functools.partial(
    pl.kernel,
    mesh=plsc.VectorSubcoreMesh(core_axis_name="c", subcore_axis_name="s"),
    out_type=jax.ShapeDtypeStruct((B, C), jnp.float32),
    scratch_types=[
        pltpu.VMEM((RPW,), jnp.int32),
        pltpu.VMEM((RPW, C), jnp.float32),
        pltpu.SemaphoreType.DMA,
    ],
)(_sc_gather)


def _mul_kernel(scales_ref, mask_t_ref, x_ref, out_ref, loss_ref):
    @pl.when(pl.program_id(0) == 0)
    def _():
        l1 = jnp.sum(jnp.abs(mask_t_ref[...]))
        loss_ref[0, 0] = jnp.maximum(l1 - LOSS_OFFSET, 0.0)

    out_ref[...] = x_ref[...] * scales_ref[...][None, :, :]


def kernel(x, labels, mask):
    xt = jnp.transpose(x, (2, 3, 0, 1)).reshape(HW, B, C)
    mask_t = mask.T

    scales = _sc_gather_call(mask_t, labels)

    out_t, loss = pl.pallas_call(
        _mul_kernel,
        grid=(NBLK,),
        in_specs=[
            pl.BlockSpec(memory_space=pltpu.VMEM),
            pl.BlockSpec((NCLS, C), lambda i: (0, 0)),
            pl.BlockSpec((HBLK, B, C), lambda i: (i, 0, 0)),
        ],
        out_specs=[
            pl.BlockSpec((HBLK, B, C), lambda i: (i, 0, 0)),
            pl.BlockSpec(memory_space=pltpu.SMEM),
        ],
        out_shape=[
            jax.ShapeDtypeStruct((HW, B, C), x.dtype),
            jax.ShapeDtypeStruct((1, 1), jnp.float32),
        ],
    )(scales, mask_t, xt)
    out = jnp.transpose(out_t.reshape(H, W, B, C), (2, 3, 0, 1))
    return out, loss[0, 0]


# speedup vs baseline: 1.2679x; 1.0283x over previous
import functools

import jax
import jax.numpy as jnp
from jax import lax
from jax.experimental import pallas as pl
from jax.experimental.pallas import tpu as pltpu
from jax.experimental.pallas import tpu_sc as plsc

B, C, H, W = 64, 768, 14, 14
HW = H * W
NCLS = 1000
LOSS_OFFSET = C * NCLS * 0.2

HBLK = 14
NBLK = HW // HBLK

GW = 8
RPW = B // GW


def _sc_gather(mask_t_hbm, labels_hbm, scales_hbm, idx_v, rows_v, sem):
    cid = lax.axis_index("c")
    sid = lax.axis_index("s")

    @pl.when((cid == 0) & (sid < GW))
    def _():
        base = sid * RPW
        pltpu.sync_copy(labels_hbm.at[pl.ds(base, RPW)], idx_v)
        pltpu.async_copy(mask_t_hbm.at[idx_v], rows_v, sem).wait()
        pltpu.sync_copy(rows_v, scales_hbm.at[pl.ds(base, RPW), :])


_sc_gather_call = functools.partial(
    pl.kernel,
    mesh=plsc.VectorSubcoreMesh(core_axis_name="c", subcore_axis_name="s", num_cores=1),
    out_type=jax.ShapeDtypeStruct((B, C), jnp.float32),
    scratch_types=[
        pltpu.VMEM((RPW,), jnp.int32),
        pltpu.VMEM((RPW, C), jnp.float32),
        pltpu.SemaphoreType.DMA,
    ],
)(_sc_gather)


def _mul_kernel(scales_ref, mask_t_ref, x_ref, out_ref, loss_ref):
    @pl.when(pl.program_id(0) == 0)
    def _():
        l1 = jnp.sum(jnp.abs(mask_t_ref[...]))
        loss_ref[0, 0] = jnp.maximum(l1 - LOSS_OFFSET, 0.0)

    out_ref[...] = x_ref[...] * scales_ref[...][None, :, :]


def kernel(x, labels, mask):
    xt = jnp.transpose(x, (2, 3, 0, 1)).reshape(HW, B, C)
    mask_t = mask.T

    scales = _sc_gather_call(mask_t, labels)

    out_t, loss = pl.pallas_call(
        _mul_kernel,
        grid=(NBLK,),
        in_specs=[
            pl.BlockSpec(memory_space=pltpu.VMEM),
            pl.BlockSpec((NCLS, C), lambda i: (0, 0)),
            pl.BlockSpec((HBLK, B, C), lambda i: (i, 0, 0)),
        ],
        out_specs=[
            pl.BlockSpec((HBLK, B, C), lambda i: (i, 0, 0)),
            pl.BlockSpec(memory_space=pltpu.SMEM),
        ],
        out_shape=[
            jax.ShapeDtypeStruct((HW, B, C), x.dtype),
            jax.ShapeDtypeStruct((1, 1), jnp.float32),
        ],
    )(scales, mask_t, xt)
    out = jnp.transpose(out_t.reshape(H, W, B, C), (2, 3, 0, 1))
    return out, loss[0, 0]


# speedup vs baseline: 1.3124x; 1.0352x over previous
import functools

import jax
import jax.numpy as jnp
from jax import lax
from jax.experimental import pallas as pl
from jax.experimental.pallas import tpu as pltpu
from jax.experimental.pallas import tpu_sc as plsc

B, C, H, W = 64, 768, 14, 14
HW = H * W
NCLS = 1000
LOSS_OFFSET = C * NCLS * 0.2

HBLK = 28
NBLK = HW // HBLK

GW = 8
RPW = B // GW


def _sc_gather(mask_t_hbm, labels_hbm, scales_hbm, idx_v, rows_v, sem):
    cid = lax.axis_index("c")
    sid = lax.axis_index("s")

    @pl.when((cid == 0) & (sid < GW))
    def _():
        base = sid * RPW
        pltpu.sync_copy(labels_hbm.at[pl.ds(base, RPW)], idx_v)
        pltpu.async_copy(mask_t_hbm.at[idx_v], rows_v, sem).wait()
        pltpu.sync_copy(rows_v, scales_hbm.at[pl.ds(base, RPW), :])


_sc_gather_call = functools.partial(
    pl.kernel,
    mesh=plsc.VectorSubcoreMesh(core_axis_name="c", subcore_axis_name="s", num_cores=1),
    out_type=jax.ShapeDtypeStruct((B, C), jnp.float32),
    scratch_types=[
        pltpu.VMEM((RPW,), jnp.int32),
        pltpu.VMEM((RPW, C), jnp.float32),
        pltpu.SemaphoreType.DMA,
    ],
)(_sc_gather)


def _mul_kernel(scales_ref, mask_t_ref, x_ref, out_ref, loss_ref):
    @pl.when(pl.program_id(0) == 0)
    def _():
        l1 = jnp.sum(jnp.abs(mask_t_ref[...]))
        loss_ref[0, 0] = jnp.maximum(l1 - LOSS_OFFSET, 0.0)

    out_ref[...] = x_ref[...] * scales_ref[...][None, :, :]


def kernel(x, labels, mask):
    xt = jnp.transpose(x, (2, 3, 0, 1)).reshape(HW, B, C)
    mask_t = mask.T

    scales = _sc_gather_call(mask_t, labels)

    out_t, loss = pl.pallas_call(
        _mul_kernel,
        grid=(NBLK,),
        in_specs=[
            pl.BlockSpec(memory_space=pltpu.VMEM),
            pl.BlockSpec((NCLS, C), lambda i: (0, 0)),
            pl.BlockSpec((HBLK, B, C), lambda i: (i, 0, 0)),
        ],
        out_specs=[
            pl.BlockSpec((HBLK, B, C), lambda i: (i, 0, 0)),
            pl.BlockSpec(memory_space=pltpu.SMEM),
        ],
        out_shape=[
            jax.ShapeDtypeStruct((HW, B, C), x.dtype),
            jax.ShapeDtypeStruct((1, 1), jnp.float32),
        ],
    )(scales, mask_t, xt)
    out = jnp.transpose(out_t.reshape(H, W, B, C), (2, 3, 0, 1))
    return out, loss[0, 0]


# speedup vs baseline: 1.3670x; 1.0416x over previous
import functools

import jax
import jax.numpy as jnp
from jax import lax
from jax.experimental import pallas as pl
from jax.experimental.pallas import tpu as pltpu
from jax.experimental.pallas import tpu_sc as plsc

B, C, H, W = 64, 768, 14, 14
HW = H * W
NCLS = 1000
LOSS_OFFSET = C * NCLS * 0.2

HBLK = 49
NBLK = HW // HBLK

GW = 8
RPW = B // GW


def _sc_gather(mask_t_hbm, labels_hbm, scales_hbm, idx_v, rows_v, sem):
    cid = lax.axis_index("c")
    sid = lax.axis_index("s")

    @pl.when((cid == 0) & (sid < GW))
    def _():
        base = sid * RPW
        pltpu.sync_copy(labels_hbm.at[pl.ds(base, RPW)], idx_v)
        pltpu.async_copy(mask_t_hbm.at[idx_v], rows_v, sem).wait()
        pltpu.sync_copy(rows_v, scales_hbm.at[pl.ds(base, RPW), :])


_sc_gather_call = functools.partial(
    pl.kernel,
    mesh=plsc.VectorSubcoreMesh(core_axis_name="c", subcore_axis_name="s", num_cores=1),
    out_type=jax.ShapeDtypeStruct((B, C), jnp.float32),
    scratch_types=[
        pltpu.VMEM((RPW,), jnp.int32),
        pltpu.VMEM((RPW, C), jnp.float32),
        pltpu.SemaphoreType.DMA,
    ],
)(_sc_gather)


def _mul_kernel(scales_ref, mask_t_ref, x_ref, out_ref, loss_ref):
    @pl.when(pl.program_id(0) == 0)
    def _():
        l1 = jnp.sum(jnp.abs(mask_t_ref[...]))
        loss_ref[0, 0] = jnp.maximum(l1 - LOSS_OFFSET, 0.0)

    out_ref[...] = x_ref[...] * scales_ref[...][None, :, :]


def kernel(x, labels, mask):
    xt = jnp.transpose(x, (2, 3, 0, 1)).reshape(HW, B, C)
    mask_t = mask.T

    scales = _sc_gather_call(mask_t, labels)

    out_t, loss = pl.pallas_call(
        _mul_kernel,
        grid=(NBLK,),
        in_specs=[
            pl.BlockSpec(memory_space=pltpu.VMEM),
            pl.BlockSpec((NCLS, C), lambda i: (0, 0)),
            pl.BlockSpec((HBLK, B, C), lambda i: (i, 0, 0)),
        ],
        out_specs=[
            pl.BlockSpec((HBLK, B, C), lambda i: (i, 0, 0)),
            pl.BlockSpec(memory_space=pltpu.SMEM),
        ],
        out_shape=[
            jax.ShapeDtypeStruct((HW, B, C), x.dtype),
            jax.ShapeDtypeStruct((1, 1), jnp.float32),
        ],
    )(scales, mask_t, xt)
    out = jnp.transpose(out_t.reshape(H, W, B, C), (2, 3, 0, 1))
    return out, loss[0, 0]


# speedup vs baseline: 1.3708x; 1.0028x over previous
import functools

import jax
import jax.numpy as jnp
from jax import lax
from jax.experimental import pallas as pl
from jax.experimental.pallas import tpu as pltpu
from jax.experimental.pallas import tpu_sc as plsc

B, C, H, W = 64, 768, 14, 14
HW = H * W
NCLS = 1000
LOSS_OFFSET = C * NCLS * 0.2

HBLK = 49
NBLK = HW // HBLK

GW = 8
RPW = B // GW


def _sc_gather(mask_t_hbm, labels_hbm, scales_hbm, idx_v, rows_v, sem):
    cid = lax.axis_index("c")
    sid = lax.axis_index("s")

    @pl.when((cid == 0) & (sid < GW))
    def _():
        base = sid * RPW
        pltpu.sync_copy(labels_hbm.at[pl.ds(base, RPW)], idx_v)
        pltpu.async_copy(mask_t_hbm.at[idx_v], rows_v, sem).wait()
        pltpu.sync_copy(rows_v, scales_hbm.at[pl.ds(base, RPW), :])


_sc_gather_call = functools.partial(
    pl.kernel,
    mesh=plsc.VectorSubcoreMesh(core_axis_name="c", subcore_axis_name="s", num_cores=1),
    out_type=jax.ShapeDtypeStruct((B, C), jnp.float32),
    scratch_types=[
        pltpu.VMEM((RPW,), jnp.int32),
        pltpu.VMEM((RPW, C), jnp.float32),
        pltpu.SemaphoreType.DMA,
    ],
)(_sc_gather)


def _mul_kernel(scales_ref, mask_t_ref, x_ref, out_ref, loss_ref):
    @pl.when(pl.program_id(0) == 0)
    def _():
        l1 = jnp.sum(jnp.abs(mask_t_ref[...]))
        loss_ref[0, 0] = jnp.maximum(l1 - LOSS_OFFSET, 0.0)

    out_ref[...] = x_ref[...] * scales_ref[...][None, :, :]


def kernel(x, labels, mask):
    xt = jnp.transpose(x, (2, 3, 0, 1)).reshape(HW, B, C)
    mask_t = mask.T

    scales = _sc_gather_call(mask_t, labels)

    out_t, loss = pl.pallas_call(
        _mul_kernel,
        grid=(NBLK,),
        in_specs=[
            pl.BlockSpec(memory_space=pltpu.VMEM),
            pl.BlockSpec((NCLS, C), lambda i: (0, 0)),
            pl.BlockSpec((HBLK, B, C), lambda i: (i, 0, 0)),
        ],
        out_specs=[
            pl.BlockSpec((HBLK, B, C), lambda i: (i, 0, 0)),
            pl.BlockSpec(memory_space=pltpu.SMEM),
        ],
        out_shape=[
            jax.ShapeDtypeStruct((HW, B, C), x.dtype),
            jax.ShapeDtypeStruct((1, 1), jnp.float32),
        ],
        compiler_params=pltpu.CompilerParams(
            dimension_semantics=("arbitrary",)),
    )(scales, mask_t, xt)
    out = jnp.transpose(out_t.reshape(H, W, B, C), (2, 3, 0, 1))
    return out, loss[0, 0]


# speedup vs baseline: 2.4099x; 1.7579x over previous
import jax
import jax.numpy as jnp
from jax.experimental import pallas as pl
from jax.experimental.pallas import tpu as pltpu

B, C, H, W = 64, 768, 14, 14
HW = H * W
NCLS = 1000
LOSS_OFFSET = C * NCLS * 0.2

HBLK = 49
NBLK = HW // HBLK


def _fused_kernel(labels_ref, mask_t_ref, x_ref, out_ref, loss_ref, scales_ref):
    @pl.when(pl.program_id(0) == 0)
    def _():
        labels_v = labels_ref[...]
        mask_t = mask_t_ref[...]
        iota = jax.lax.broadcasted_iota(jnp.int32, (B, NCLS), 1)
        onehot = (iota == labels_v[:, None]).astype(jnp.float32)
        scales_ref[...] = jax.lax.dot_general(
            onehot, mask_t,
            dimension_numbers=(((1,), (0,)), ((), ())),
            preferred_element_type=jnp.float32,
        )
        l1 = jnp.sum(jnp.abs(mask_t))
        loss_ref[0, 0] = jnp.maximum(l1 - LOSS_OFFSET, 0.0)

    out_ref[...] = x_ref[...] * scales_ref[...][None, :, :]


def kernel(x, labels, mask):
    xt = jnp.transpose(x, (2, 3, 0, 1)).reshape(HW, B, C)
    mask_t = mask.T
    out_t, loss = pl.pallas_call(
        _fused_kernel,
        grid=(NBLK,),
        in_specs=[
            pl.BlockSpec(memory_space=pltpu.VMEM),
            pl.BlockSpec((NCLS, C), lambda i: (0, 0)),
            pl.BlockSpec((HBLK, B, C), lambda i: (i, 0, 0)),
        ],
        out_specs=[
            pl.BlockSpec((HBLK, B, C), lambda i: (i, 0, 0)),
            pl.BlockSpec(memory_space=pltpu.SMEM),
        ],
        out_shape=[
            jax.ShapeDtypeStruct((HW, B, C), x.dtype),
            jax.ShapeDtypeStruct((1, 1), jnp.float32),
        ],
        scratch_shapes=[pltpu.VMEM((B, C), jnp.float32)],
    )(labels, mask_t, xt)
    out = jnp.transpose(out_t.reshape(H, W, B, C), (2, 3, 0, 1))
    return out, loss[0, 0]
